# baseline (speedup 1.0000x reference)
import functools

import jax
import jax.numpy as jnp
from jax import lax
from jax.experimental import pallas as pl
from jax.experimental.pallas import tpu as pltpu
from jax.experimental.pallas import tpu_sc as plsc

N_EXPERTS = 64
TOPK = 8
BLOCK_TOK = 1024

NC = 2
NS = 16
NW = NC * NS
SUB = 512

INTERPRET = False


def _gate_block(w_ref, b_ref, x_ref, out_ref):
    logits = (
        jax.lax.dot_general(
            w_ref[:],
            x_ref[:],
            (((1,), (1,)), ((), ())),
            preferred_element_type=jnp.float32,
        )
        + b_ref[:]
    )
    out_ref[:] = logits.T


def _gate_logits(x, W, b):
    n_tokens = x.shape[0]
    d_model = x.shape[1]
    b2 = b.reshape(N_EXPERTS, 1)
    grid = (n_tokens // BLOCK_TOK,)
    return pl.pallas_call(
        _gate_block,
        grid=grid,
        in_specs=[
            pl.BlockSpec((N_EXPERTS, d_model), lambda i: (0, 0)),
            pl.BlockSpec((N_EXPERTS, 1), lambda i: (0, 0)),
            pl.BlockSpec((BLOCK_TOK, d_model), lambda i: (i, 0)),
        ],
        out_specs=pl.BlockSpec((BLOCK_TOK, N_EXPERTS), lambda i: (i, 0)),
        out_shape=jax.ShapeDtypeStruct((n_tokens, N_EXPERTS), jnp.float32),
        interpret=INTERPRET,
    )(W, b2, x)


def _merge_top8(ak, av, bk, bv, lo8):
    ck = jnp.where(lo8, ak, lax.rev(bk, (0,)))
    cv = jnp.where(lo8, av, lax.rev(bv, (0,)))
    return plsc.sort_key_val(ck, cv, descending=True)


def _sc_router(logits_flat, n_tokens):
    tpw = n_tokens // NW
    sub = min(SUB, tpw)
    mesh = plsc.VectorSubcoreMesh(
        core_axis_name="c", subcore_axis_name="s", num_cores=NC, num_subcores=NS
    )

    @functools.partial(
        pl.kernel,
        out_type=[
            jax.ShapeDtypeStruct((n_tokens * N_EXPERTS,), jnp.float32),
            jax.ShapeDtypeStruct((n_tokens * 16,), jnp.int32),
        ],
        mesh=mesh,
        scratch_types=[
            pltpu.VMEM((sub * N_EXPERTS,), jnp.float32),
            pltpu.VMEM((sub * N_EXPERTS,), jnp.float32),
            pltpu.VMEM((sub * 16,), jnp.int32),
        ],
        interpret=INTERPRET,
        compiler_params=pltpu.CompilerParams(needs_layout_passes=False),
    )
    def sc_kernel(logits_hbm, probs_hbm, idx_hbm, lchunk, pout, iout):
        wid = lax.axis_index("s") * NC + lax.axis_index("c")
        iota = lax.iota(jnp.int32, 16)
        lo8 = iota < 8
        zeros = jnp.zeros((16,), jnp.float32)

        @pl.loop(0, tpw // sub)
        def _chunk(ci):
            base = wid * tpw + ci * sub

            pltpu.sync_copy(
                logits_hbm.at[pl.ds(base * N_EXPERTS, sub * N_EXPERTS)], lchunk
            )

            @plsc.parallel_loop(0, sub, unroll=8)
            def _tok(t):
                o = t * N_EXPERTS
                k0 = lchunk[pl.ds(o, 16)]
                k1 = lchunk[pl.ds(o + 16, 16)]
                k2 = lchunk[pl.ds(o + 32, 16)]
                k3 = lchunk[pl.ds(o + 48, 16)]
                a0k, a0v = plsc.sort_key_val(k0, iota, descending=True)
                a1k, a1v = plsc.sort_key_val(k1, iota + 16, descending=True)
                a2k, a2v = plsc.sort_key_val(k2, iota + 32, descending=True)
                a3k, a3v = plsc.sort_key_val(k3, iota + 48, descending=True)
                s1k, s1v = _merge_top8(a0k, a0v, a1k, a1v, lo8)
                s2k, s2v = _merge_top8(a2k, a2v, a3k, a3v, lo8)
                s3k, s3v = _merge_top8(s1k, s1v, s2k, s2v, lo8)
                m = jnp.max(s3k)
                e = jnp.where(lo8, jnp.exp(s3k - m), 0.0)
                p = e / jnp.sum(e)
                pout[pl.ds(o, 16)] = zeros
                pout[pl.ds(o + 16, 16)] = zeros
                pout[pl.ds(o + 32, 16)] = zeros
                pout[pl.ds(o + 48, 16)] = zeros
                plsc.store_scatter(pout, [o + s3v], p, mask=lo8)
                iout[pl.ds(t * 16, 16)] = s3v

            pltpu.sync_copy(
                pout, probs_hbm.at[pl.ds(base * N_EXPERTS, sub * N_EXPERTS)]
            )
            pltpu.sync_copy(iout, idx_hbm.at[pl.ds(base * 16, sub * 16)])

    return sc_kernel(logits_flat)


N_CHUNKS = 4


@jax.jit
def kernel(x, W, b):
    n_tokens = x.shape[0]
    ct = n_tokens // N_CHUNKS
    probs_parts = []
    idx_parts = []
    for c in range(N_CHUNKS):
        xc = jax.lax.slice_in_dim(x, c * ct, (c + 1) * ct, axis=0)
        logits = _gate_logits(xc, W, b)
        probs_flat, idx_flat = _sc_router(logits.reshape(-1), ct)
        probs_parts.append(probs_flat.reshape(ct, N_EXPERTS))
        idx_parts.append(idx_flat.reshape(ct, 16)[:, :TOPK])
    return (
        jnp.concatenate(probs_parts, axis=0),
        jnp.concatenate(idx_parts, axis=0),
    )


# speedup vs baseline: 2.4304x; 2.4304x over previous
import functools

import jax
import jax.numpy as jnp
from jax import lax
from jax.experimental import pallas as pl
from jax.experimental.pallas import tpu as pltpu
from jax.experimental.pallas import tpu_sc as plsc

N_EXPERTS = 64
TOPK = 8
BLOCK_TOK = 1024

NC = 2
NS = 16
NW = NC * NS
SUB = 512

INTERPRET = False


def _gate_block(w_ref, b_ref, x_ref, out_ref):
    logits = (
        jax.lax.dot_general(
            w_ref[:],
            x_ref[:],
            (((1,), (1,)), ((), ())),
            preferred_element_type=jnp.float32,
        )
        + b_ref[:]
    )
    out_ref[:] = logits.T


def _gate_logits(x, W, b):
    n_tokens = x.shape[0]
    d_model = x.shape[1]
    b2 = b.reshape(N_EXPERTS, 1)
    grid = (n_tokens // BLOCK_TOK,)
    return pl.pallas_call(
        _gate_block,
        grid=grid,
        in_specs=[
            pl.BlockSpec((N_EXPERTS, d_model), lambda i: (0, 0)),
            pl.BlockSpec((N_EXPERTS, 1), lambda i: (0, 0)),
            pl.BlockSpec((BLOCK_TOK, d_model), lambda i: (i, 0)),
        ],
        out_specs=pl.BlockSpec((BLOCK_TOK, N_EXPERTS), lambda i: (i, 0)),
        out_shape=jax.ShapeDtypeStruct((n_tokens, N_EXPERTS), jnp.float32),
        interpret=INTERPRET,
    )(W, b2, x)


def _merge_top8(ak, av, bk, bv, lo8):
    ck = jnp.where(lo8, ak, lax.rev(bk, (0,)))
    cv = jnp.where(lo8, av, lax.rev(bv, (0,)))
    return plsc.sort_key_val(ck, cv, descending=True)


def _sc_router(logits_flat, n_tokens):
    tpw = n_tokens // NW
    sub = min(SUB, tpw)
    mesh = plsc.VectorSubcoreMesh(
        core_axis_name="c", subcore_axis_name="s", num_cores=NC, num_subcores=NS
    )

    @functools.partial(
        pl.kernel,
        out_type=[
            jax.ShapeDtypeStruct((n_tokens * N_EXPERTS,), jnp.float32),
            jax.ShapeDtypeStruct((n_tokens * 16,), jnp.int32),
        ],
        mesh=mesh,
        scratch_types=[
            pltpu.VMEM((sub * N_EXPERTS,), jnp.float32),
            pltpu.VMEM((sub * N_EXPERTS,), jnp.float32),
            pltpu.VMEM((sub * 16,), jnp.int32),
        ],
        interpret=INTERPRET,
        compiler_params=pltpu.CompilerParams(needs_layout_passes=False),
    )
    def sc_kernel(logits_hbm, probs_hbm, idx_hbm, lchunk, pout, iout):
        wid = lax.axis_index("s") * NC + lax.axis_index("c")
        iota = lax.iota(jnp.int32, 16)
        lo8 = iota < 8
        zeros = jnp.zeros((16,), jnp.float32)

        @pl.loop(0, tpw // sub)
        def _chunk(ci):
            base = wid * tpw + ci * sub

            pltpu.sync_copy(
                logits_hbm.at[pl.ds(base * N_EXPERTS, sub * N_EXPERTS)], lchunk
            )

            @plsc.parallel_loop(0, sub, unroll=8)
            def _tok(t):
                o = t * N_EXPERTS
                k0 = lchunk[pl.ds(o, 16)]
                k1 = lchunk[pl.ds(o + 16, 16)]
                k2 = lchunk[pl.ds(o + 32, 16)]
                k3 = lchunk[pl.ds(o + 48, 16)]
                a0k, a0v = plsc.sort_key_val(k0, iota, descending=True)
                a1k, a1v = plsc.sort_key_val(k1, iota + 16, descending=True)
                a2k, a2v = plsc.sort_key_val(k2, iota + 32, descending=True)
                a3k, a3v = plsc.sort_key_val(k3, iota + 48, descending=True)
                s1k, s1v = _merge_top8(a0k, a0v, a1k, a1v, lo8)
                s2k, s2v = _merge_top8(a2k, a2v, a3k, a3v, lo8)
                s3k, s3v = _merge_top8(s1k, s1v, s2k, s2v, lo8)
                m = jnp.max(s3k)
                e = jnp.where(lo8, jnp.exp(s3k - m), 0.0)
                p = e / jnp.sum(e)
                pout[pl.ds(o, 16)] = zeros
                pout[pl.ds(o + 16, 16)] = zeros
                pout[pl.ds(o + 32, 16)] = zeros
                pout[pl.ds(o + 48, 16)] = zeros
                plsc.store_scatter(pout, [o + s3v], p, mask=lo8)
                iout[pl.ds(t * 16, 16)] = s3v

            pltpu.sync_copy(
                pout, probs_hbm.at[pl.ds(base * N_EXPERTS, sub * N_EXPERTS)]
            )
            pltpu.sync_copy(iout, idx_hbm.at[pl.ds(base * 16, sub * 16)])

    return sc_kernel(logits_flat)


N_CHUNKS = 1


@jax.jit
def kernel(x, W, b):
    n_tokens = x.shape[0]
    ct = n_tokens // N_CHUNKS
    probs_parts = []
    idx_parts = []
    for c in range(N_CHUNKS):
        xc = jax.lax.slice_in_dim(x, c * ct, (c + 1) * ct, axis=0)
        logits = _gate_logits(xc, W, b)
        probs_flat, idx_flat = _sc_router(logits.reshape(-1), ct)
        probs_parts.append(probs_flat.reshape(ct, N_EXPERTS))
        idx_parts.append(idx_flat.reshape(ct, 16)[:, :TOPK])
    return (
        jnp.concatenate(probs_parts, axis=0),
        jnp.concatenate(idx_parts, axis=0),
    )


# speedup vs baseline: 2.4574x; 1.0111x over previous
import functools

import jax
import jax.numpy as jnp
from jax import lax
from jax.experimental import pallas as pl
from jax.experimental.pallas import tpu as pltpu
from jax.experimental.pallas import tpu_sc as plsc

N_EXPERTS = 64
TOPK = 8
BLOCK_TOK = 1024

NC = 2
NS = 16
NW = NC * NS
SUB = 512


def _gate_block(w_ref, b_ref, x_ref, out_ref):
    logits = (
        jax.lax.dot_general(
            w_ref[:],
            x_ref[:],
            (((1,), (1,)), ((), ())),
            preferred_element_type=jnp.float32,
        )
        + b_ref[:]
    )
    out_ref[:] = logits.T


def _gate_logits(x, W, b):
    n_tokens = x.shape[0]
    d_model = x.shape[1]
    b2 = b.reshape(N_EXPERTS, 1)
    grid = (n_tokens // BLOCK_TOK,)
    return pl.pallas_call(
        _gate_block,
        grid=grid,
        in_specs=[
            pl.BlockSpec((N_EXPERTS, d_model), lambda i: (0, 0)),
            pl.BlockSpec((N_EXPERTS, 1), lambda i: (0, 0)),
            pl.BlockSpec((BLOCK_TOK, d_model), lambda i: (i, 0)),
        ],
        out_specs=pl.BlockSpec((BLOCK_TOK, N_EXPERTS), lambda i: (i, 0)),
        out_shape=jax.ShapeDtypeStruct((n_tokens, N_EXPERTS), jnp.float32),
    )(W, b2, x)


def _merge_top8(ak, av, bk, bv, lo8):
    ck = jnp.where(lo8, ak, lax.rev(bk, (0,)))
    cv = jnp.where(lo8, av, lax.rev(bv, (0,)))
    return plsc.sort_key_val(ck, cv, descending=True)


def _sc_router(logits_flat, n_tokens):
    tpw = n_tokens // NW
    sub = min(SUB, tpw)
    mesh = plsc.VectorSubcoreMesh(
        core_axis_name="c", subcore_axis_name="s", num_cores=NC, num_subcores=NS
    )

    @functools.partial(
        pl.kernel,
        out_type=[
            jax.ShapeDtypeStruct((n_tokens * N_EXPERTS,), jnp.float32),
            jax.ShapeDtypeStruct((n_tokens * 16,), jnp.int32),
        ],
        mesh=mesh,
        scratch_types=[
            pltpu.VMEM((sub * N_EXPERTS,), jnp.float32),
            pltpu.VMEM((sub * N_EXPERTS,), jnp.float32),
            pltpu.VMEM((sub * 16,), jnp.int32),
        ],
        compiler_params=pltpu.CompilerParams(needs_layout_passes=False),
    )
    def sc_kernel(logits_hbm, probs_hbm, idx_hbm, lchunk, pout, iout):
        wid = lax.axis_index("s") * NC + lax.axis_index("c")
        iota = lax.iota(jnp.int32, 16)
        lo8 = iota < 8
        zeros = jnp.zeros((16,), jnp.float32)

        @pl.loop(0, tpw // sub)
        def _chunk(ci):
            base = wid * tpw + ci * sub

            pltpu.sync_copy(
                logits_hbm.at[pl.ds(base * N_EXPERTS, sub * N_EXPERTS)], lchunk
            )

            @plsc.parallel_loop(0, sub, unroll=4)
            def _tok(t):
                o = t * N_EXPERTS
                k0 = lchunk[pl.ds(o, 16)]
                k1 = lchunk[pl.ds(o + 16, 16)]
                k2 = lchunk[pl.ds(o + 32, 16)]
                k3 = lchunk[pl.ds(o + 48, 16)]
                a0k, a0v = plsc.sort_key_val(k0, iota, descending=True)
                a1k, a1v = plsc.sort_key_val(k1, iota + 16, descending=True)
                a2k, a2v = plsc.sort_key_val(k2, iota + 32, descending=True)
                a3k, a3v = plsc.sort_key_val(k3, iota + 48, descending=True)
                s1k, s1v = _merge_top8(a0k, a0v, a1k, a1v, lo8)
                s2k, s2v = _merge_top8(a2k, a2v, a3k, a3v, lo8)
                s3k, s3v = _merge_top8(s1k, s1v, s2k, s2v, lo8)
                m = jnp.max(s3k)
                e = jnp.where(lo8, jnp.exp(s3k - m), 0.0)
                p = e / jnp.sum(e)
                pout[pl.ds(o, 16)] = zeros
                pout[pl.ds(o + 16, 16)] = zeros
                pout[pl.ds(o + 32, 16)] = zeros
                pout[pl.ds(o + 48, 16)] = zeros
                plsc.store_scatter(pout, [o + s3v], p, mask=lo8)
                iout[pl.ds(t * 16, 16)] = s3v

            pltpu.sync_copy(
                pout, probs_hbm.at[pl.ds(base * N_EXPERTS, sub * N_EXPERTS)]
            )
            pltpu.sync_copy(iout, idx_hbm.at[pl.ds(base * 16, sub * 16)])

    return sc_kernel(logits_flat)


@jax.jit
def kernel(x, W, b):
    n_tokens = x.shape[0]
    logits = _gate_logits(x, W, b)
    probs_flat, idx_flat = _sc_router(logits.reshape(-1), n_tokens)
    probs = probs_flat.reshape(n_tokens, N_EXPERTS)
    idx = idx_flat.reshape(n_tokens, 16)[:, :TOPK]
    return (probs, idx)
